# Initial kernel scaffold; baseline (speedup 1.0000x reference)
import jax
import jax.numpy as jnp
from jax.experimental import pallas as pl


def kernel(emb, nbr_idx, xy, Wq, Wk, Wv, We1, be1, We2, be2):
    raise NotImplementedError("write your pallas kernel here")


# speedup vs baseline: 1.6894x; 1.6894x over previous
import functools
import math

import jax
import jax.numpy as jnp
from jax import lax
from jax.experimental import pallas as pl
from jax.experimental.pallas import tpu as pltpu
from jax.experimental.pallas import tpu_sc as plsc

N = 10000
K_NBR = 32
HID = 128
ATTN = 128
SCALE = 1.0 / math.sqrt(ATTN)

NWORKERS = 32
ROWS_PER_BATCH = 4
EDGES_PER_BATCH = ROWS_PER_BATCH * K_NBR
NBATCH = N // ROWS_PER_BATCH
STEPS = (NBATCH + NWORKERS - 1) // NWORKERS

_f32 = jnp.float32
_i32 = jnp.int32


def _dense_body(e_ref, wq_ref, wk_ref, wv_ref, we2_ref,
                qs_ref, g_ref, kv_ref):
    e = e_ref[...]
    q = jnp.dot(e, wq_ref[...], preferred_element_type=_f32)
    qs_ref[...] = q * SCALE
    g_ref[...] = lax.dot_general(
        q, we2_ref[...], (((1,), (1,)), ((), ())),
        preferred_element_type=_f32) * SCALE
    kv_ref[:, :ATTN] = jnp.dot(e, wk_ref[...], preferred_element_type=_f32)
    kv_ref[:, ATTN:] = jnp.dot(e, wv_ref[...], preferred_element_type=_f32)


def _dense_stage(emb, Wq, Wk, Wv, We2):
    blk = 1000
    grid = N // blk
    full = lambda shape: pl.BlockSpec(shape, lambda i: (0, 0))
    return pl.pallas_call(
        _dense_body,
        grid=(grid,),
        in_specs=[
            pl.BlockSpec((blk, HID), lambda i: (i, 0)),
            full((HID, ATTN)), full((HID, ATTN)), full((HID, HID)),
            full((ATTN, ATTN)),
        ],
        out_specs=[
            pl.BlockSpec((blk, ATTN), lambda i: (i, 0)),
            pl.BlockSpec((blk, ATTN), lambda i: (i, 0)),
            pl.BlockSpec((blk, ATTN + HID), lambda i: (i, 0)),
        ],
        out_shape=[
            jax.ShapeDtypeStruct((N, ATTN), _f32),
            jax.ShapeDtypeStruct((N, ATTN), _f32),
            jax.ShapeDtypeStruct((N, ATTN + HID), _f32),
        ],
    )(emb, Wq, Wk, Wv, We2)


def _fast_rsqrt(x):
    xi = plsc.bitcast(x, _i32)
    yi = jnp.int32(0x5F3759DF) - lax.shift_right_logical(xi, 1)
    y = plsc.bitcast(yi, _f32)
    for _ in range(3):
        y = y * (1.5 - 0.5 * x * y * y)
    return y


def _sc_body(kv_hbm, qs_hbm, g_hbm, emb_hbm, nbrf_hbm, xyf_hbm, w_hbm,
             out_hbm,
             xy_v, w_v, idx_v, kv_v, q_v, g_v, e_v, out_v, wgt_v,
             d_v, s_v, c_v, sem):
    wid = lax.axis_index("s") * 2 + lax.axis_index("c")

    pltpu.sync_copy(xyf_hbm, xy_v)
    pltpu.sync_copy(w_hbm, w_v)

    iota16 = lax.broadcasted_iota(_i32, (16,), 0)
    zero16 = jnp.zeros((16,), _i32)
    one16 = jnp.ones((16,), _i32)

    def splat(v):
        return jnp.full((16,), v, _i32)

    def do_batch(b):
        base = b * ROWS_PER_BATCH
        pltpu.sync_copy(nbrf_hbm.at[pl.ds(base * K_NBR, EDGES_PER_BATCH)],
                        idx_v)
        pltpu.sync_copy(qs_hbm.at[pl.ds(base, ROWS_PER_BATCH)], q_v)
        pltpu.sync_copy(g_hbm.at[pl.ds(base, ROWS_PER_BATCH)], g_v)
        pltpu.sync_copy(emb_hbm.at[pl.ds(base, ROWS_PER_BATCH)], e_v)
        pltpu.async_copy(kv_hbm.at[idx_v], kv_v, sem).wait()

        for gi in range(EDGES_PER_BATCH // 16):
            j16 = idx_v[pl.ds(gi * 16, 16)]
            j2 = j16 + j16
            x16 = plsc.load_gather(xy_v, [j2])
            y16 = plsc.load_gather(xy_v, [j2 + 1])
            own2 = splat(2 * (base + (gi * 16) // K_NBR))
            xi = plsc.load_gather(xy_v, [own2])
            yi = plsc.load_gather(xy_v, [own2 + 1])
            dx = x16 - xi
            dy = y16 - yi
            r2 = dx * dx + dy * dy
            rs = _fast_rsqrt(r2)
            valid = r2 > 0.0
            d_v[pl.ds(gi * 16, 16)] = jnp.where(valid, r2 * rs, 0.0)
            s_v[pl.ds(gi * 16, 16)] = jnp.where(valid, dy * rs, 0.0)
            c_v[pl.ds(gi * 16, 16)] = jnp.where(valid, dx * rs, 1.0)

        for r in range(ROWS_PER_BATCH):
            c0acc = jnp.zeros((16,), _f32)
            for ch in range(8):
                c0acc = c0acc + (q_v[r, pl.ds(ch * 16, 16)]
                                 * w_v[4, pl.ds(ch * 16, 16)])
            c0 = jnp.sum(c0acc)

            halves = []
            for half in range(2):
                eb = (2 * r + half) * 16
                d16 = d_v[pl.ds(eb, 16)]
                s16 = s_v[pl.ds(eb, 16)]
                c16 = c_v[pl.ds(eb, 16)]
                eidx = eb + iota16

                def mbody(m, acc, _d=d16, _s=s16, _c=c16, _e=eidx, _r=r):
                    m16 = splat(m)
                    r16 = splat(_r)
                    k16 = plsc.load_gather(kv_v, [_e, m16])
                    w0 = plsc.load_gather(w_v, [zero16, m16])
                    w1 = plsc.load_gather(w_v, [one16, m16])
                    w2 = plsc.load_gather(w_v, [splat(2), m16])
                    b1 = plsc.load_gather(w_v, [splat(3), m16])
                    qm = plsc.load_gather(q_v, [r16, m16])
                    gm = plsc.load_gather(g_v, [r16, m16])
                    h = jnp.maximum(_d * w0 + _s * w1 + _c * w2 + b1, 0.0)
                    return acc + qm * k16 + gm * h

                acc = lax.fori_loop(0, ATTN, mbody, jnp.zeros((16,), _f32),
                                    unroll=4)
                halves.append(acc + c0)

            sA, sB = halves
            mx = jnp.max(jnp.maximum(sA, sB))
            ea = jnp.exp(sA - mx)
            eb2 = jnp.exp(sB - mx)
            den16 = jnp.zeros((16,), _f32) + jnp.sum(ea + eb2)
            winv = jnp.ones((16,), _f32) / den16
            wgt_v[pl.ds(r * K_NBR, 16)] = ea * winv
            wgt_v[pl.ds(r * K_NBR + 16, 16)] = eb2 * winv

            def ebody(e, accs, _r=r):
                row = _r * K_NBR + e
                we = plsc.load_gather(wgt_v, [splat(row)])
                return tuple(
                    accs[ch] + we * kv_v[row, pl.ds(ATTN + ch * 16, 16)]
                    for ch in range(8))

            accs = lax.fori_loop(
                0, K_NBR, ebody,
                tuple(jnp.zeros((16,), _f32) for _ in range(8)), unroll=2)
            for ch in range(8):
                out_v[r, pl.ds(ch * 16, 16)] = (accs[ch]
                                                + e_v[r, pl.ds(ch * 16, 16)])

        pltpu.sync_copy(out_v, out_hbm.at[pl.ds(base, ROWS_PER_BATCH)])

    def step(t, carry):
        b = wid + NWORKERS * t

        @pl.when(b < NBATCH)
        def _():
            do_batch(b)

        return carry

    lax.fori_loop(0, STEPS, step, 0)


def _sc_stage(kv, qs, g, emb, nbrf, xyf, wcat):
    mesh = plsc.VectorSubcoreMesh(core_axis_name="c", subcore_axis_name="s")
    f = pl.kernel(
        _sc_body,
        out_type=jax.ShapeDtypeStruct((N, HID), _f32),
        mesh=mesh,
        compiler_params=pltpu.CompilerParams(needs_layout_passes=False),
        scratch_types=[
            pltpu.VMEM((2 * N,), _f32),
            pltpu.VMEM((5, ATTN), _f32),
            pltpu.VMEM((EDGES_PER_BATCH,), _i32),
            pltpu.VMEM((EDGES_PER_BATCH, ATTN + HID), _f32),
            pltpu.VMEM((ROWS_PER_BATCH, ATTN), _f32),
            pltpu.VMEM((ROWS_PER_BATCH, ATTN), _f32),
            pltpu.VMEM((ROWS_PER_BATCH, HID), _f32),
            pltpu.VMEM((ROWS_PER_BATCH, HID), _f32),
            pltpu.VMEM((EDGES_PER_BATCH,), _f32),
            pltpu.VMEM((EDGES_PER_BATCH,), _f32),
            pltpu.VMEM((EDGES_PER_BATCH,), _f32),
            pltpu.VMEM((EDGES_PER_BATCH,), _f32),
            pltpu.SemaphoreType.DMA,
        ],
    )
    return f(kv, qs, g, emb, nbrf, xyf, wcat)


def kernel(emb, nbr_idx, xy, Wq, Wk, Wv, We1, be1, We2, be2):
    qs, g, kv = _dense_stage(emb, Wq, Wk, Wv, We2)
    wcat = jnp.stack([We1[0], We1[1], We1[2], be1, be2])
    nbrf = nbr_idx.reshape(-1)
    return _sc_stage(kv, qs, g, emb, nbrf, xy.reshape(-1), wcat)


# speedup vs baseline: 3.5502x; 2.1015x over previous
import functools
import math

import jax
import jax.numpy as jnp
from jax import lax
from jax.experimental import pallas as pl
from jax.experimental.pallas import tpu as pltpu
from jax.experimental.pallas import tpu_sc as plsc

N = 10000
K_NBR = 32
HID = 128
ATTN = 128
SCALE = 1.0 / math.sqrt(ATTN)

NWORKERS = 32
ROWS_PER_BATCH = 4
EDGES_PER_BATCH = ROWS_PER_BATCH * K_NBR
NBATCH = N // ROWS_PER_BATCH
STEPS = (NBATCH + NWORKERS - 1) // NWORKERS
NCH = ATTN // 16

_f32 = jnp.float32
_i32 = jnp.int32


def _dense_body(e_ref, wq_ref, wk_ref, wv_ref, we2_ref,
                qge_ref, kv_ref):
    e = e_ref[...]
    q = jnp.dot(e, wq_ref[...], preferred_element_type=_f32)
    qge_ref[:, :ATTN] = q * SCALE
    qge_ref[:, ATTN:2 * ATTN] = lax.dot_general(
        q, we2_ref[...], (((1,), (1,)), ((), ())),
        preferred_element_type=_f32) * SCALE
    qge_ref[:, 2 * ATTN:] = e
    kv_ref[:, :ATTN] = jnp.dot(e, wk_ref[...], preferred_element_type=_f32)
    kv_ref[:, ATTN:] = jnp.dot(e, wv_ref[...], preferred_element_type=_f32)


def _dense_stage(emb, Wq, Wk, Wv, We2):
    blk = 1000
    grid = N // blk
    full = lambda shape: pl.BlockSpec(shape, lambda i: (0, 0))
    return pl.pallas_call(
        _dense_body,
        grid=(grid,),
        in_specs=[
            pl.BlockSpec((blk, HID), lambda i: (i, 0)),
            full((HID, ATTN)), full((HID, ATTN)), full((HID, HID)),
            full((ATTN, ATTN)),
        ],
        out_specs=[
            pl.BlockSpec((blk, 2 * ATTN + HID), lambda i: (i, 0)),
            pl.BlockSpec((blk, ATTN + HID), lambda i: (i, 0)),
        ],
        out_shape=[
            jax.ShapeDtypeStruct((N, 2 * ATTN + HID), _f32),
            jax.ShapeDtypeStruct((N, ATTN + HID), _f32),
        ],
    )(emb, Wq, Wk, Wv, We2)


def _fast_rsqrt(x):
    xi = plsc.bitcast(x, _i32)
    yi = jnp.int32(0x5F3759DF) - lax.shift_right_logical(xi, 1)
    y = plsc.bitcast(yi, _f32)
    for _ in range(3):
        y = y * (1.5 - 0.5 * x * y * y)
    return y


def _sc_body(kv_hbm, qge_hbm, nbrf_hbm, xyf_hbm, w_hbm,
             out_hbm,
             xy_v, w_v, idx_v, kv_v, qge_v, out_v, wgt_v,
             d_v, s_v, c_v, sem):
    wid = lax.axis_index("s") * 2 + lax.axis_index("c")

    pltpu.sync_copy(xyf_hbm, xy_v)
    pltpu.sync_copy(w_hbm, w_v)

    iota16 = lax.broadcasted_iota(_i32, (16,), 0)

    def splat(v):
        return jnp.full((16,), v, _i32)

    wv0 = [w_v[0, pl.ds(ch * 16, 16)] for ch in range(NCH)]
    wv1 = [w_v[1, pl.ds(ch * 16, 16)] for ch in range(NCH)]
    wv2 = [w_v[2, pl.ds(ch * 16, 16)] for ch in range(NCH)]
    bv1 = [w_v[3, pl.ds(ch * 16, 16)] for ch in range(NCH)]
    be2 = [w_v[4, pl.ds(ch * 16, 16)] for ch in range(NCH)]

    def do_batch(b):
        base = b * ROWS_PER_BATCH
        pltpu.sync_copy(nbrf_hbm.at[pl.ds(base * K_NBR, EDGES_PER_BATCH)],
                        idx_v)
        pltpu.sync_copy(qge_hbm.at[pl.ds(base, ROWS_PER_BATCH)], qge_v)
        pltpu.async_copy(kv_hbm.at[idx_v], kv_v, sem).wait()

        for gi in range(EDGES_PER_BATCH // 16):
            j16 = idx_v[pl.ds(gi * 16, 16)]
            j2 = j16 + j16
            x16 = plsc.load_gather(xy_v, [j2])
            y16 = plsc.load_gather(xy_v, [j2 + 1])
            own2 = splat(2 * (base + (gi * 16) // K_NBR))
            xi = plsc.load_gather(xy_v, [own2])
            yi = plsc.load_gather(xy_v, [own2 + 1])
            dx = x16 - xi
            dy = y16 - yi
            r2 = dx * dx + dy * dy
            rs = _fast_rsqrt(r2)
            valid = r2 > 0.0
            d_v[pl.ds(gi * 16, 16)] = jnp.where(valid, r2 * rs, 0.0)
            s_v[pl.ds(gi * 16, 16)] = jnp.where(valid, dy * rs, 0.0)
            c_v[pl.ds(gi * 16, 16)] = jnp.where(valid, dx * rs, 1.0)

        def row_body(r, carry):
            rbase = r * K_NBR
            qc = [qge_v[r, pl.ds(ch * 16, 16)] for ch in range(NCH)]
            gc = [qge_v[r, pl.ds(ATTN + ch * 16, 16)] for ch in range(NCH)]

            c0acc = qc[0] * be2[0]
            for ch in range(1, NCH):
                c0acc = c0acc + qc[ch] * be2[ch]
            c016 = jnp.zeros((16,), _f32) + jnp.sum(c0acc)

            def score_half(half):
                svec = c016
                for e in range(16):
                    ei = rbase + half * 16 + e
                    ei16 = splat(ei)
                    dd = plsc.load_gather(d_v, [ei16])
                    ss = plsc.load_gather(s_v, [ei16])
                    cc = plsc.load_gather(c_v, [ei16])
                    acc = None
                    for ch in range(NCH):
                        k = kv_v[ei, pl.ds(ch * 16, 16)]
                        aff = dd * wv0[ch] + ss * wv1[ch] + cc * wv2[ch] \
                            + bv1[ch]
                        h = jnp.maximum(aff, 0.0)
                        term = qc[ch] * k + gc[ch] * h
                        acc = term if ch == 0 else acc + term
                    se16 = jnp.zeros((16,), _f32) + jnp.sum(acc)
                    svec = jnp.where(iota16 == e, se16, svec)
                return svec

            sA = score_half(0)
            sB = score_half(1)
            mx = jnp.max(jnp.maximum(sA, sB))
            ea = jnp.exp(sA - mx)
            eb = jnp.exp(sB - mx)
            den16 = jnp.zeros((16,), _f32) + jnp.sum(ea + eb)
            winv = jnp.ones((16,), _f32) / den16
            wgt_v[pl.ds(rbase, 16)] = ea * winv
            wgt_v[pl.ds(rbase + 16, 16)] = eb * winv

            def ebody(e, accs):
                row = rbase + e
                we = plsc.load_gather(wgt_v, [splat(row)])
                return tuple(
                    accs[ch] + we * kv_v[row, pl.ds(ATTN + ch * 16, 16)]
                    for ch in range(NCH))

            accs = lax.fori_loop(
                0, K_NBR, ebody,
                tuple(jnp.zeros((16,), _f32) for _ in range(NCH)), unroll=2)
            for ch in range(NCH):
                out_v[r, pl.ds(ch * 16, 16)] = (
                    accs[ch] + qge_v[r, pl.ds(2 * ATTN + ch * 16, 16)])
            return carry

        lax.fori_loop(0, ROWS_PER_BATCH, row_body, 0)
        pltpu.sync_copy(out_v, out_hbm.at[pl.ds(base, ROWS_PER_BATCH)])

    def step(t, carry):
        b = wid + NWORKERS * t

        @pl.when(b < NBATCH)
        def _():
            do_batch(b)

        return carry

    lax.fori_loop(0, STEPS, step, 0)


def _sc_stage(kv, qge, nbrf, xyf, wcat):
    mesh = plsc.VectorSubcoreMesh(core_axis_name="c", subcore_axis_name="s")
    f = pl.kernel(
        _sc_body,
        out_type=jax.ShapeDtypeStruct((N, HID), _f32),
        mesh=mesh,
        compiler_params=pltpu.CompilerParams(needs_layout_passes=False),
        scratch_types=[
            pltpu.VMEM((2 * N,), _f32),
            pltpu.VMEM((5, ATTN), _f32),
            pltpu.VMEM((EDGES_PER_BATCH,), _i32),
            pltpu.VMEM((EDGES_PER_BATCH, ATTN + HID), _f32),
            pltpu.VMEM((ROWS_PER_BATCH, 2 * ATTN + HID), _f32),
            pltpu.VMEM((ROWS_PER_BATCH, HID), _f32),
            pltpu.VMEM((EDGES_PER_BATCH,), _f32),
            pltpu.VMEM((EDGES_PER_BATCH,), _f32),
            pltpu.VMEM((EDGES_PER_BATCH,), _f32),
            pltpu.VMEM((EDGES_PER_BATCH,), _f32),
            pltpu.SemaphoreType.DMA,
        ],
    )
    return f(kv, qge, nbrf, xyf, wcat)


def kernel(emb, nbr_idx, xy, Wq, Wk, Wv, We1, be1, We2, be2):
    qge, kv = _dense_stage(emb, Wq, Wk, Wv, We2)
    wcat = jnp.stack([We1[0], We1[1], We1[2], be1, be2])
    nbrf = nbr_idx.reshape(-1)
    return _sc_stage(kv, qge, nbrf, xy.reshape(-1), wcat)
